# baseline (speedup 1.0000x reference)
import jax
import jax.numpy as jnp
from jax.experimental import pallas as pl

_B, _N, _D, _K = 32, 1000, 128, 16
_M = _N - 1
_E = _M * _K + 2 * _M
_LB = 1024
_RB = 8
_NRB = 125
_IMAX = 2**31 - 1


def _topk_kernel(csub_ref, clane_ref, dep_ref, vals_ref, idx_ref, d0_ref):
    xl = clane_ref[0, 0:1, :]
    yl = clane_ref[0, 1:2, :]
    dxd = xl - dep_ref[0, :, 0:1]
    dyd = yl - dep_ref[0, :, 1:2]
    sqd = dxd * dxd + dyd * dyd
    d0_ref[0] = jnp.where(sqd > 0, jnp.sqrt(jnp.where(sqd > 0, sqd, 1.0)), 0.0)

    lane = jax.lax.broadcasted_iota(jnp.int32, (_RB, _LB), 1)

    def body(rb, carry):
        i0 = rb * _RB
        xi = csub_ref[0, pl.ds(i0, _RB), 0:1]
        yi = csub_ref[0, pl.ds(i0, _RB), 1:2]
        dx = xi - xl
        dy = yi - yl
        sq = dx * dx + dy * dy
        row_ids = i0 + jax.lax.broadcasted_iota(jnp.int32, (_RB, _LB), 0)
        key = jnp.bitwise_or(
            jnp.bitwise_and(jax.lax.bitcast_convert_type(sq, jnp.int32),
                            -1024),
            lane)
        key = jnp.where(lane == row_ids, _IMAX, key)
        mins = []
        for _ in range(_K):
            kmin = jnp.min(key, axis=1, keepdims=True)
            key = jnp.where(key == kmin, _IMAX, key)
            mins.append(kmin)
        kk = jnp.concatenate(mins, axis=1)
        idx = jnp.bitwise_and(kk, 1023)
        sqv = jax.lax.bitcast_convert_type(
            jnp.bitwise_and(kk, -1024), jnp.float32)
        vals_ref[0, pl.ds(i0, _RB), :] = jnp.sqrt(sqv)
        idx_ref[0, pl.ds(i0, _RB), :] = idx
        return carry

    jax.lax.fori_loop(0, _NRB, body, 0)


def _expand_kernel(attr_ref, w_ref, b_ref, out_ref):
    out_ref[...] = attr_ref[...] * w_ref[...] + b_ref[...]


def kernel(locs, init_embeddings, W, b):
    B, N, D, K, M, E, LB = _B, _N, _D, _K, _M, _E, _LB
    cust = locs[:, 1:, :]
    pad = jnp.full((B, LB - M, 2), 1e6, jnp.float32)
    csub = jnp.concatenate([cust, pad], axis=1)
    clane = csub.transpose(0, 2, 1)
    dep = locs[:, 0:1, :]

    vals_p, idx_p, d0p = pl.pallas_call(
        _topk_kernel,
        grid=(B,),
        in_specs=[
            pl.BlockSpec((1, LB, 2), lambda i: (i, 0, 0)),
            pl.BlockSpec((1, 2, LB), lambda i: (i, 0, 0)),
            pl.BlockSpec((1, 1, 2), lambda i: (i, 0, 0)),
        ],
        out_specs=[
            pl.BlockSpec((1, 1000, K), lambda i: (i, 0, 0)),
            pl.BlockSpec((1, 1000, K), lambda i: (i, 0, 0)),
            pl.BlockSpec((1, 1, LB), lambda i: (i, 0, 0)),
        ],
        out_shape=[
            jax.ShapeDtypeStruct((B, 1000, K), jnp.float32),
            jax.ShapeDtypeStruct((B, 1000, K), jnp.int32),
            jax.ShapeDtypeStruct((B, 1, LB), jnp.float32),
        ],
    )(csub, clane, dep)

    vals = vals_p[:, :M, :]
    idx = idx_p[:, :M, :]
    d0 = d0p[:, 0, :M]

    attr_flat = jnp.concatenate([vals.reshape(B, M * K), d0, d0], axis=1)
    attr_col = attr_flat.reshape(B * E, 1)
    edge_attr = pl.pallas_call(
        _expand_kernel,
        grid=(999,),
        in_specs=[
            pl.BlockSpec((576, 1), lambda e: (e, 0)),
            pl.BlockSpec((1, D), lambda e: (0, 0)),
            pl.BlockSpec((1, D), lambda e: (0, 0)),
        ],
        out_specs=pl.BlockSpec((576, D), lambda e: (e, 0)),
        out_shape=jax.ShapeDtypeStruct((B * E, D), jnp.float32),
    )(attr_col, W.reshape(1, D), b.reshape(1, D))

    cust_ids = jnp.arange(1, N, dtype=jnp.int32)
    src_b = jnp.concatenate(
        [jnp.repeat(cust_ids, K), cust_ids, jnp.zeros((M,), jnp.int32)])
    dst_b = jnp.concatenate(
        [(idx + 1).reshape(B, M * K),
         jnp.zeros((B, M), jnp.int32),
         jnp.broadcast_to(cust_ids[None, :], (B, M))], axis=1)
    off = (jnp.arange(B, dtype=jnp.int32) * N)[:, None]
    src = jnp.broadcast_to(src_b[None, :] + off, (B, E)).reshape(-1)
    dst = (dst_b + off).reshape(-1)
    edge_index = jnp.stack([src, dst])

    x = init_embeddings.reshape(B * N, D)
    return (x, edge_index, edge_attr)


# speedup vs baseline: 9.5084x; 9.5084x over previous
import functools

import jax
import jax.numpy as jnp
from jax import lax
from jax.experimental import pallas as pl
from jax.experimental.pallas import tpu as pltpu
from jax.experimental.pallas import tpu_sc as plsc

_B, _N, _D, _K = 32, 1000, 128, 16
_M = _N - 1
_E = _M * _K + 2 * _M
_LB = 1024
_RB = 16
_NCH = 64
_NRB = 1


def _topk_kernel(csub_ref, clane_ref, dep_ref, vals_ref, idx_ref, d0_ref):
    xl = clane_ref[0, 0:1, :]
    yl = clane_ref[0, 1:2, :]
    dxd = xl - dep_ref[0, :, 0:1]
    dyd = yl - dep_ref[0, :, 1:2]
    sqd = dxd * dxd + dyd * dyd
    d0_ref[0] = jnp.where(sqd > 0, jnp.sqrt(jnp.where(sqd > 0, sqd, 1.0)), 0.0)

    lane = jax.lax.broadcasted_iota(jnp.int32, (_RB, _LB), 1)

    def build_key(i0):
        xi = csub_ref[0, pl.ds(i0, _RB), 0:1]
        yi = csub_ref[0, pl.ds(i0, _RB), 1:2]
        dx = xi - xl
        dy = yi - yl
        sq = dx * dx + dy * dy
        row_ids = i0 + jax.lax.broadcasted_iota(jnp.int32, (_RB, _LB), 0)
        kbits = jnp.bitwise_or(
            jnp.bitwise_and(jax.lax.bitcast_convert_type(sq, jnp.int32),
                            -1024),
            lane)
        key = jax.lax.bitcast_convert_type(kbits, jnp.float32)
        return jnp.where(lane == row_ids, jnp.inf, key)

    def store_topk(i0, mins):
        kk = jax.lax.bitcast_convert_type(
            jnp.concatenate(mins, axis=1), jnp.int32)
        idx = jnp.bitwise_and(kk, 1023)
        sqv = jax.lax.bitcast_convert_type(
            jnp.bitwise_and(kk, -1024), jnp.float32)
        vals_ref[0, pl.ds(i0, _RB), :] = jnp.sqrt(sqv)
        idx_ref[0, pl.ds(i0, _RB), :] = idx

    def body(rb, carry):
        offs = [(rb * _NCH + j) * _RB for j in range(_NCH)]
        keys = [build_key(i0) for i0 in offs]
        mins = [[] for _ in offs]
        for _ in range(_K):
            for j in range(_NCH):
                kmin = jnp.min(keys[j], axis=1, keepdims=True)
                keys[j] = jnp.where(keys[j] == kmin, jnp.inf, keys[j])
                mins[j].append(kmin)
        for j, i0 in enumerate(offs):
            store_topk(i0, mins[j])
        return carry

    jax.lax.fori_loop(0, _NRB, body, 0)


def _expand_kernel(attr_ref, w_ref, b_ref, out_ref):
    out_ref[...] = attr_ref[...] * w_ref[...] + b_ref[...]


_EP = 18000


def _make_ei_kernel():
    mesh = plsc.VectorSubcoreMesh(core_axis_name="c", subcore_axis_name="s")

    @functools.partial(
        pl.kernel, mesh=mesh,
        out_type=jax.ShapeDtypeStruct((2, _B, _EP), jnp.int32),
        scratch_types=[
            pltpu.VMEM((_LB * _K,), jnp.int32),
            pltpu.VMEM((_EP,), jnp.int32),
            pltpu.VMEM((_EP,), jnp.int32),
        ],
    )
    def ei_kernel(idx_hbm, out_hbm, idx_v, src_v, dst_v):
        b = lax.axis_index("s") * 2 + lax.axis_index("c")
        pltpu.sync_copy(idx_hbm.at[b], idx_v)
        off = b * _N
        iota = lax.broadcasted_iota(jnp.int32, (16,), 0)

        def knn_body(c, carry):
            p = c * 16
            dst_v[pl.ds(p, 16)] = idx_v[pl.ds(p, 16)] + (off + 1)
            src_v[pl.ds(p, 16)] = jnp.zeros((16,), jnp.int32) + (off + 1 + c)
            return carry

        lax.fori_loop(0, _M, knn_body, 0)

        def dep_body(t, carry):
            r = t * 16 + iota
            in_first = r < _M
            src_v[pl.ds(_M * _K + t * 16, 16)] = jnp.where(
                in_first, off + 1 + r, off)
            dst_v[pl.ds(_M * _K + t * 16, 16)] = jnp.where(
                in_first, jnp.zeros((16,), jnp.int32) + off, off + 1 + (r - _M))
            return carry

        lax.fori_loop(0, (_EP - _M * _K) // 16, dep_body, 0)
        pltpu.sync_copy(src_v, out_hbm.at[0, b])
        pltpu.sync_copy(dst_v, out_hbm.at[1, b])

    return ei_kernel


def kernel(locs, init_embeddings, W, b):
    B, N, D, K, M, E, LB = _B, _N, _D, _K, _M, _E, _LB
    cust = locs[:, 1:, :]
    pad = jnp.full((B, LB - M, 2), 1e6, jnp.float32)
    csub = jnp.concatenate([cust, pad], axis=1)
    clane = csub.transpose(0, 2, 1)
    dep = locs[:, 0:1, :]

    vals_p, idx_p, d0p = pl.pallas_call(
        _topk_kernel,
        grid=(B,),
        in_specs=[
            pl.BlockSpec((1, LB, 2), lambda i: (i, 0, 0)),
            pl.BlockSpec((1, 2, LB), lambda i: (i, 0, 0)),
            pl.BlockSpec((1, 1, 2), lambda i: (i, 0, 0)),
        ],
        out_specs=[
            pl.BlockSpec((1, LB, K), lambda i: (i, 0, 0)),
            pl.BlockSpec((1, LB, K), lambda i: (i, 0, 0)),
            pl.BlockSpec((1, 1, LB), lambda i: (i, 0, 0)),
        ],
        out_shape=[
            jax.ShapeDtypeStruct((B, LB, K), jnp.float32),
            jax.ShapeDtypeStruct((B, LB, K), jnp.int32),
            jax.ShapeDtypeStruct((B, 1, LB), jnp.float32),
        ],
    )(csub, clane, dep)

    vals = vals_p[:, :M, :]
    d0 = d0p[:, 0, :M]

    ei_pad = _make_ei_kernel()(idx_p.reshape(B, LB * K))
    edge_index = ei_pad[:, :, :E].reshape(2, B * E)

    attr_flat = jnp.concatenate([vals.reshape(B, M * K), d0, d0], axis=1)
    attr_col = attr_flat.reshape(B * E, 1)
    edge_attr = pl.pallas_call(
        _expand_kernel,
        grid=(999,),
        in_specs=[
            pl.BlockSpec((576, 1), lambda e: (e, 0)),
            pl.BlockSpec((1, D), lambda e: (0, 0)),
            pl.BlockSpec((1, D), lambda e: (0, 0)),
        ],
        out_specs=pl.BlockSpec((576, D), lambda e: (e, 0)),
        out_shape=jax.ShapeDtypeStruct((B * E, D), jnp.float32),
    )(attr_col, W.reshape(1, D), b.reshape(1, D))

    x = init_embeddings.reshape(B * N, D)
    return (x, edge_index, edge_attr)
